# Initial kernel scaffold; baseline (speedup 1.0000x reference)
import jax
import jax.numpy as jnp
from jax.experimental import pallas as pl


def kernel(x, edge_index, batch, params):
    raise NotImplementedError("write your pallas kernel here")


# speedup vs baseline: 1.2273x; 1.2273x over previous
import jax, jax.numpy as jnp
from jax.experimental import pallas as pl

N, E, F, H, L, NE, C, G = 10000, 320000, 128, 64, 3, 4, 10, 128


def kernel(x, edge_index, batch, params):
    src, dst = edge_index[0], edge_index[1]

    def scatter(rows, idx, n):
        return jnp.zeros((n, rows.shape[1]), jnp.float32).at[idx].add(rows)

    h = x
    xW = None
    for l in range(L):
        W1, b1 = params['enc_W1_%d' % l], params['enc_b1_%d' % l]
        W2, b2 = params['enc_W2_%d' % l], params['enc_b2_%d' % l]
        eps = params['enc_eps_%d' % l]
        hW = h @ W1
        agg = scatter(hW[src], dst, N)
        z1 = jnp.maximum((1.0 + eps) * hW + agg + b1, 0.0)
        h = jnp.maximum(z1 @ W2 + b2, 0.0)
        if l == 0:
            xW = hW
    Z = h

    counts = jnp.maximum(scatter(jnp.ones((N, 1), jnp.float32), batch, G), 1.0)
    h_orig = scatter(Z, batch, G) / counts

    nW1cat = jnp.concatenate([params['node_W1'][e] for e in range(NE)], axis=1)
    nb1cat = jnp.concatenate([params['node_b1'][e] for e in range(NE)])
    T = jnp.maximum(Z @ nW1cat + nb1cat, 0.0)
    nw2 = params['node_W2'][:, :, 0].reshape(NE * H)
    nm = jax.nn.sigmoid((T * nw2).reshape(N, NE, H).sum(-1) + params['node_b2'][:, 0])

    eW1a = jnp.concatenate([params['edge_W1'][e, :H, :] for e in range(NE)], axis=1)
    eW1b = jnp.concatenate([params['edge_W1'][e, H:, :] for e in range(NE)], axis=1)
    A = Z @ eW1a
    B = Z @ eW1b
    eb1cat = params['edge_b1'].reshape(NE * H)
    U = jnp.maximum(A[src] + B[dst] + eb1cat, 0.0)
    ew2 = params['edge_W2'][:, :, 0].reshape(NE * H)
    em = jax.nn.sigmoid((U * ew2).reshape(E, NE, H).sum(-1) + params['edge_b2'][:, 0])

    def blockdiag(W):
        return jnp.kron(jnp.eye(NE, dtype=W.dtype), W)

    w0 = em * nm[src]
    eps0 = params['enc_eps_0']
    msg0 = jnp.repeat(w0, H, axis=1) * jnp.tile(xW[src], (1, NE))
    agg0 = scatter(msg0, dst, N)
    self0 = jnp.repeat(nm, H, axis=1) * jnp.tile(xW, (1, NE))
    b1t = jnp.tile(params['enc_b1_0'], NE)
    z1 = jnp.maximum((1.0 + eps0) * self0 + agg0 + b1t, 0.0)
    hm = jnp.maximum(z1 @ blockdiag(params['enc_W2_0']) + jnp.tile(params['enc_b2_0'], NE), 0.0)
    emr = jnp.repeat(em, H, axis=1)
    for l in range(1, L):
        W1, b1 = params['enc_W1_%d' % l], params['enc_b1_%d' % l]
        W2, b2 = params['enc_W2_%d' % l], params['enc_b2_%d' % l]
        eps = params['enc_eps_%d' % l]
        hW = hm @ blockdiag(W1)
        agg = scatter(emr * hW[src], dst, N)
        z1 = jnp.maximum((1.0 + eps) * hW + agg + jnp.tile(b1, NE), 0.0)
        hm = jnp.maximum(z1 @ blockdiag(W2) + jnp.tile(b2, NE), 0.0)

    h_st = (scatter(hm, batch, G) / counts).reshape(G, NE, H)
    logits = jnp.einsum('geh,ehc->gec', h_st, params['cls_W']) + params['cls_b']
    return (logits, h_st, h_orig, nm[:, :, None], em[:, :, None])


# speedup vs baseline: 1.9450x; 1.5848x over previous
import functools
import jax, jax.numpy as jnp
from jax import lax
from jax.experimental import pallas as pl
from jax.experimental.pallas import tpu as pltpu, tpu_sc as plsc

N, E, F, H, L, NE, C, G = 10000, 320000, 128, 64, 3, 4, 10, 128
NC, NS = 2, 16
NW = NC * NS
K = 80


def _bcast16(val):
    return jnp.zeros((16,), jnp.float32) + val


def _fulli(k):
    return jnp.zeros((16,), jnp.int32) + k


_NROWCH = N // 80 + (1 if N % 80 else 0)


def _zero_shared(zbuf, acc, s, w):
    zv = jnp.zeros((16,), jnp.float32)
    for i in range(16):
        for d in range(w // 16):
            zbuf[i, pl.ds(d * 16, 16)] = zv
    for t in range(8):
        ch = s * 8 + t

        @pl.when(ch < _NROWCH)
        def _():
            for q in range(5):
                r0 = pl.multiple_of(ch * 80 + q * 16, 16)
                pltpu.sync_copy(zbuf, acc.at[pl.ds(r0, 16)])


def _copy_out(acc, out_slice, s):
    for t in range(8):
        ch = s * 8 + t

        @pl.when(ch < _NROWCH)
        def _():
            r0 = pl.multiple_of(ch * 80, 16)
            pltpu.sync_copy(acc.at[pl.ds(r0, 80)], out_slice.at[pl.ds(r0, 80)])


def _agg_plain(table, src, dst):
    EW = E // NW
    NCH = EW // K
    mesh = plsc.VectorSubcoreMesh(core_axis_name="c", subcore_axis_name="s")

    @functools.partial(
        pl.kernel,
        out_type=jax.ShapeDtypeStruct((NC, N, H), jnp.float32),
        mesh=mesh,
        compiler_params=pltpu.CompilerParams(use_tc_tiling_on_sc=False, needs_layout_passes=False),
        scratch_types=[
            pltpu.VMEM((K,), jnp.int32),
            pltpu.VMEM((K,), jnp.int32),
            pltpu.VMEM((K, H), jnp.float32),
            pltpu.VMEM((16, H), jnp.float32),
            pltpu.VMEM_SHARED((N, H), jnp.float32),
            pltpu.SemaphoreType.DMA,
        ],
    )
    def k(table_hbm, src_hbm, dst_hbm, out_hbm, sidx, didx, rows, zbuf, acc, sem):
        c = lax.axis_index("c")
        s = lax.axis_index("s")
        wid = s * NC + c
        _zero_shared(zbuf, acc, s, H)
        plsc.subcore_barrier()

        @pl.loop(0, NCH)
        def body(j):
            base = pl.multiple_of(wid * EW + j * K, 16)
            pltpu.sync_copy(src_hbm.at[pl.ds(base, K)], sidx)
            pltpu.sync_copy(dst_hbm.at[pl.ds(base, K)], didx)
            pltpu.async_copy(table_hbm.at[sidx], rows, sem).wait()
            pltpu.sync_copy(rows, acc.at[didx], add=True)

        plsc.subcore_barrier()
        _copy_out(acc, out_hbm.at[c], s)

    return k(table, src, dst)


def _agg_pair(table, src, dst, w):
    EW = E // NS
    NCH = EW // K
    W2 = 2 * H
    mesh = plsc.VectorSubcoreMesh(core_axis_name="c", subcore_axis_name="s")

    @functools.partial(
        pl.kernel,
        out_type=jax.ShapeDtypeStruct((NC, N, W2), jnp.float32),
        mesh=mesh,
        compiler_params=pltpu.CompilerParams(use_tc_tiling_on_sc=False, needs_layout_passes=False),
        scratch_types=[
            pltpu.VMEM((K,), jnp.int32),
            pltpu.VMEM((K,), jnp.int32),
            pltpu.VMEM((K, W2), jnp.float32),
            pltpu.VMEM((K, W2), jnp.float32),
            pltpu.VMEM((K, NE), jnp.float32),
            pltpu.VMEM((16, W2), jnp.float32),
            pltpu.VMEM_SHARED((N, W2), jnp.float32),
            pltpu.SemaphoreType.DMA,
        ],
    )
    def k(table_hbm, src_hbm, dst_hbm, w_hbm, out_hbm,
          sidx, didx, rows, prod, wbuf, zbuf, acc, sem):
        c = lax.axis_index("c")
        s = lax.axis_index("s")
        _zero_shared(zbuf, acc, s, W2)
        plsc.subcore_barrier()

        @pl.loop(0, NCH)
        def body(j):
            base = pl.multiple_of(s * EW + j * K, 16)
            pltpu.sync_copy(src_hbm.at[pl.ds(base, K)], sidx)
            pltpu.sync_copy(dst_hbm.at[pl.ds(base, K)], didx)
            pltpu.sync_copy(w_hbm.at[pl.ds(base, K)], wbuf)
            pltpu.async_copy(table_hbm.at[c].at[sidx], rows, sem).wait()

            @pl.loop(0, K)
            def edge(e):
                ke = _fulli(e)
                w0 = plsc.load_gather(wbuf, [ke, _fulli(2 * c)])
                w1 = plsc.load_gather(wbuf, [ke, _fulli(2 * c + 1)])
                for d in range(H // 16):
                    prod[e, pl.ds(d * 16, 16)] = rows[e, pl.ds(d * 16, 16)] * w0
                for d in range(H // 16):
                    prod[e, pl.ds(H + d * 16, 16)] = rows[e, pl.ds(H + d * 16, 16)] * w1

            pltpu.sync_copy(prod, acc.at[didx], add=True)

        plsc.subcore_barrier()
        _copy_out(acc, out_hbm.at[c], s)

    return k(table, src, dst, w)


def _agg_pair0(table, src, dst, w):
    TW = 80
    EW = E // NS
    NCH = EW // K
    W2 = 2 * H
    mesh = plsc.VectorSubcoreMesh(core_axis_name="c", subcore_axis_name="s")

    @functools.partial(
        pl.kernel,
        out_type=jax.ShapeDtypeStruct((NC, N, W2), jnp.float32),
        mesh=mesh,
        compiler_params=pltpu.CompilerParams(use_tc_tiling_on_sc=False, needs_layout_passes=False),
        scratch_types=[
            pltpu.VMEM((K,), jnp.int32),
            pltpu.VMEM((K,), jnp.int32),
            pltpu.VMEM((K, TW), jnp.float32),
            pltpu.VMEM((K, W2), jnp.float32),
            pltpu.VMEM((K, NE), jnp.float32),
            pltpu.VMEM((16, W2), jnp.float32),
            pltpu.VMEM_SHARED((N, W2), jnp.float32),
            pltpu.SemaphoreType.DMA,
        ],
    )
    def k(table_hbm, src_hbm, dst_hbm, w_hbm, out_hbm,
          sidx, didx, rows, prod, wbuf, zbuf, acc, sem):
        c = lax.axis_index("c")
        s = lax.axis_index("s")
        _zero_shared(zbuf, acc, s, W2)
        plsc.subcore_barrier()

        @pl.loop(0, NCH)
        def body(j):
            base = pl.multiple_of(s * EW + j * K, 16)
            pltpu.sync_copy(src_hbm.at[pl.ds(base, K)], sidx)
            pltpu.sync_copy(dst_hbm.at[pl.ds(base, K)], didx)
            pltpu.sync_copy(w_hbm.at[pl.ds(base, K)], wbuf)
            pltpu.async_copy(table_hbm.at[sidx], rows, sem).wait()

            @pl.loop(0, K)
            def edge(e):
                ke = _fulli(e)
                w0 = plsc.load_gather(wbuf, [ke, _fulli(2 * c)]) * \
                    plsc.load_gather(rows, [ke, _fulli(H + 2 * c)])
                w1 = plsc.load_gather(wbuf, [ke, _fulli(2 * c + 1)]) * \
                    plsc.load_gather(rows, [ke, _fulli(H + 2 * c + 1)])
                for d in range(H // 16):
                    prod[e, pl.ds(d * 16, 16)] = rows[e, pl.ds(d * 16, 16)] * w0
                for d in range(H // 16):
                    prod[e, pl.ds(H + d * 16, 16)] = rows[e, pl.ds(d * 16, 16)] * w1

            pltpu.sync_copy(prod, acc.at[didx], add=True)

        plsc.subcore_barrier()
        _copy_out(acc, out_hbm.at[c], s)

    return k(table, src, dst, w)


def kernel(x, edge_index, batch, params):
    src, dst = edge_index[0], edge_index[1]

    def scatter(rows, idx, n):
        return jnp.zeros((n, rows.shape[1]), jnp.float32).at[idx].add(rows)

    h = x
    xW = None
    for l in range(L):
        W1, b1 = params['enc_W1_%d' % l], params['enc_b1_%d' % l]
        W2, b2 = params['enc_W2_%d' % l], params['enc_b2_%d' % l]
        eps = params['enc_eps_%d' % l]
        hW = h @ W1
        p = _agg_plain(hW, src, dst)
        agg = p[0] + p[1]
        z1 = jnp.maximum((1.0 + eps) * hW + agg + b1, 0.0)
        h = jnp.maximum(z1 @ W2 + b2, 0.0)
        if l == 0:
            xW = hW
    Z = h

    counts = jnp.maximum(scatter(jnp.ones((N, 1), jnp.float32), batch, G), 1.0)
    h_orig = scatter(Z, batch, G) / counts

    nW1cat = jnp.concatenate([params['node_W1'][e] for e in range(NE)], axis=1)
    nb1cat = jnp.concatenate([params['node_b1'][e] for e in range(NE)])
    T = jnp.maximum(Z @ nW1cat + nb1cat, 0.0)
    nw2 = params['node_W2'][:, :, 0].reshape(NE * H)
    nm = jax.nn.sigmoid((T * nw2).reshape(N, NE, H).sum(-1) + params['node_b2'][:, 0])

    eW1a = jnp.concatenate([params['edge_W1'][e, :H, :] for e in range(NE)], axis=1)
    eW1b = jnp.concatenate([params['edge_W1'][e, H:, :] for e in range(NE)], axis=1)
    A = Z @ eW1a
    B = Z @ eW1b
    eb1cat = params['edge_b1'].reshape(NE * H)
    U = jnp.maximum(A[src] + B[dst] + eb1cat, 0.0)
    ew2 = params['edge_W2'][:, :, 0].reshape(NE * H)
    em = jax.nn.sigmoid((U * ew2).reshape(E, NE, H).sum(-1) + params['edge_b2'][:, 0])

    def blockdiag(W):
        return jnp.kron(jnp.eye(NE, dtype=W.dtype), W)

    def to_pair(hcat):
        return hcat.reshape(N, NC, 2 * H).transpose(1, 0, 2)

    def from_pair(p):
        return p.transpose(1, 0, 2).reshape(N, NE * H)

    eps0 = params['enc_eps_0']
    xwnm = jnp.concatenate([xW, nm, jnp.zeros((N, 80 - H - NE), jnp.float32)], axis=1)
    agg0 = from_pair(_agg_pair0(xwnm, src, dst, em))
    self0 = jnp.repeat(nm, H, axis=1) * jnp.tile(xW, (1, NE))
    b1t = jnp.tile(params['enc_b1_0'], NE)
    z1 = jnp.maximum((1.0 + eps0) * self0 + agg0 + b1t, 0.0)
    hm = jnp.maximum(z1 @ blockdiag(params['enc_W2_0']) + jnp.tile(params['enc_b2_0'], NE), 0.0)
    for l in range(1, L):
        W1, b1 = params['enc_W1_%d' % l], params['enc_b1_%d' % l]
        W2, b2 = params['enc_W2_%d' % l], params['enc_b2_%d' % l]
        eps = params['enc_eps_%d' % l]
        hW = hm @ blockdiag(W1)
        agg = from_pair(_agg_pair(to_pair(hW), src, dst, em))
        z1 = jnp.maximum((1.0 + eps) * hW + agg + jnp.tile(b1, NE), 0.0)
        hm = jnp.maximum(z1 @ blockdiag(W2) + jnp.tile(b2, NE), 0.0)

    h_st = (scatter(hm, batch, G) / counts).reshape(G, NE, H)
    logits = jnp.einsum('geh,ehc->gec', h_st, params['cls_W']) + params['cls_b']
    return (logits, h_st, h_orig, nm[:, :, None], em[:, :, None])


# speedup vs baseline: 3.3656x; 1.7304x over previous
import functools
import jax, jax.numpy as jnp
from jax import lax
from jax.experimental import pallas as pl
from jax.experimental.pallas import tpu as pltpu, tpu_sc as plsc

N, E, F, H, L, NE, C, G = 10000, 320000, 128, 64, 3, 4, 10, 128
NC, NS = 2, 16
NW = NC * NS
K = 80


def _bcast16(val):
    return jnp.zeros((16,), jnp.float32) + val


def _fulli(k):
    return jnp.zeros((16,), jnp.int32) + k


_NROWCH = N // 80 + (1 if N % 80 else 0)


def _zero_shared(zbuf, acc, s, w):
    zv = jnp.zeros((16,), jnp.float32)
    for i in range(16):
        for d in range(w // 16):
            zbuf[i, pl.ds(d * 16, 16)] = zv
    for t in range(8):
        ch = s * 8 + t

        @pl.when(ch < _NROWCH)
        def _():
            for q in range(5):
                r0 = pl.multiple_of(ch * 80 + q * 16, 16)
                pltpu.sync_copy(zbuf, acc.at[pl.ds(r0, 16)])


def _copy_out(acc, out_slice, s):
    for t in range(8):
        ch = s * 8 + t

        @pl.when(ch < _NROWCH)
        def _():
            r0 = pl.multiple_of(ch * 80, 16)
            pltpu.sync_copy(acc.at[pl.ds(r0, 80)], out_slice.at[pl.ds(r0, 80)])


def _make_agg(mode):
    W2 = 2 * H
    weighted = mode != 'plain'
    TW = H if mode == 'plain' else (W2 if mode == 'pair' else 80)
    OW = H if mode == 'plain' else W2
    per_worker = E // NW if mode == 'plain' else E // NS
    NCH = per_worker // K
    HALF = NCH // 2
    KN = K * NE
    mesh = plsc.VectorSubcoreMesh(core_axis_name="c", subcore_axis_name="s")

    scratch = [
        pltpu.VMEM((2, K), jnp.int32),
        pltpu.VMEM((2, K), jnp.int32),
        pltpu.VMEM((2, K, TW), jnp.float32),
        pltpu.VMEM((16, OW), jnp.float32),
        pltpu.VMEM_SHARED((N, OW), jnp.float32),
    ]
    if weighted:
        scratch.append(pltpu.VMEM((2, KN), jnp.float32))
    if mode == 'pair0':
        scratch.append(pltpu.VMEM((K, W2), jnp.float32))
    nsem = 8 if weighted else 6
    scratch += [pltpu.SemaphoreType.DMA] * nsem

    def body(table_hbm, src_hbm, dst_hbm, w_hbm, out_hbm, *refs):
        it = iter(refs)
        sidx, didx, rows, zbuf, acc = (next(it) for _ in range(5))
        wbuf = next(it) if weighted else None
        prod = next(it) if mode == 'pair0' else None
        sems = list(it)
        si, di, g = sems[0:2], sems[2:4], sems[4:6]
        wi = sems[6:8] if weighted else None

        c = lax.axis_index("c")
        s = lax.axis_index("s")
        worker = s * NC + c if mode == 'plain' else s

        def base_of(j):
            return pl.multiple_of(worker * per_worker + j * K, 16)

        def issue_idx(j, b):
            base = base_of(j)
            pltpu.async_copy(src_hbm.at[pl.ds(base, K)], sidx.at[b], si[b])
            pltpu.async_copy(dst_hbm.at[pl.ds(base, K)], didx.at[b], di[b])
            if weighted:
                b4 = pl.multiple_of(base * NE, 16)
                pltpu.async_copy(w_hbm.at[pl.ds(b4, KN)], wbuf.at[b], wi[b])

        def wait_si(b):
            pltpu.make_async_copy(src_hbm.at[pl.ds(0, K)], sidx.at[b], si[b]).wait()

        def wait_di(b):
            pltpu.make_async_copy(dst_hbm.at[pl.ds(0, K)], didx.at[b], di[b]).wait()

        def wait_wi(b):
            pltpu.make_async_copy(w_hbm.at[pl.ds(0, KN)], wbuf.at[b], wi[b]).wait()

        def tbl():
            return table_hbm.at[c] if mode == 'pair' else table_hbm

        def issue_gather(b):
            pltpu.async_copy(tbl().at[sidx.at[b]], rows.at[b], g[b])

        def wait_g(b):
            pltpu.make_async_copy(tbl().at[sidx.at[b]], rows.at[b], g[b]).wait()

        def compute(b):
            @pl.loop(0, K, unroll=8)
            def edge(e):
                i0 = _fulli(e * NE + 2 * c)
                w0 = plsc.load_gather(wbuf.at[b], [i0])
                w1 = plsc.load_gather(wbuf.at[b], [i0 + 1])
                if mode == 'pair0':
                    ke = _fulli(e)
                    w0 = w0 * plsc.load_gather(rows.at[b], [ke, _fulli(H + 2 * c)])
                    w1 = w1 * plsc.load_gather(rows.at[b], [ke, _fulli(H + 2 * c + 1)])
                    for d in range(H // 16):
                        r = rows[b, e, pl.ds(d * 16, 16)]
                        prod[e, pl.ds(d * 16, 16)] = r * w0
                        prod[e, pl.ds(H + d * 16, 16)] = r * w1
                else:
                    for d in range(H // 16):
                        rows[b, e, pl.ds(d * 16, 16)] = rows[b, e, pl.ds(d * 16, 16)] * w0
                    for d in range(H // 16):
                        rows[b, e, pl.ds(H + d * 16, 16)] = rows[b, e, pl.ds(H + d * 16, 16)] * w1

        _zero_shared(zbuf, acc, s, OW)
        plsc.subcore_barrier()
        issue_idx(0, 0)
        issue_idx(1, 1)
        wait_si(0)
        issue_gather(0)

        @pl.loop(0, HALF)
        def outer(mi):
            for b in (0, 1):
                j = mi * 2 + b

                @pl.when(j + 1 < NCH)
                def _():
                    wait_si(1 - b)
                    issue_gather(1 - b)

                wait_g(b)
                wait_di(b)
                if weighted:
                    wait_wi(b)
                    compute(b)
                srcbuf = prod if mode == 'pair0' else rows.at[b]
                pltpu.sync_copy(srcbuf, acc.at[didx.at[b]], add=True)

                @pl.when(j + 2 < NCH)
                def _():
                    issue_idx(j + 2, b)

        if NCH % 2:
            wait_g(0)
            wait_di(0)
            if weighted:
                wait_wi(0)
                compute(0)
            srcbuf = prod if mode == 'pair0' else rows.at[0]
            pltpu.sync_copy(srcbuf, acc.at[didx.at[0]], add=True)

        plsc.subcore_barrier()
        _copy_out(acc, out_hbm.at[c], s)

    out_t = jax.ShapeDtypeStruct((NC, N, OW), jnp.float32)
    k = functools.partial(
        pl.kernel, out_type=out_t, mesh=mesh,
        compiler_params=pltpu.CompilerParams(use_tc_tiling_on_sc=False,
                                             needs_layout_passes=False),
        scratch_types=scratch)(body)
    return k


_agg_plain_k = _make_agg('plain')
_agg_pair_k = _make_agg('pair')
_agg_pair0_k = _make_agg('pair0')
_DUMMY_W = None


def _agg_plain(table, src, dst):
    w = jnp.zeros((NE,), jnp.float32)
    return _agg_plain_k(table, src, dst, w)


def _agg_pair(table, src, dst, w):
    return _agg_pair_k(table, src, dst, w)


def _agg_pair0(table, src, dst, w):
    return _agg_pair0_k(table, src, dst, w)


def kernel(x, edge_index, batch, params):
    src, dst = edge_index[0], edge_index[1]

    def scatter(rows, idx, n):
        return jnp.zeros((n, rows.shape[1]), jnp.float32).at[idx].add(rows)

    h = x
    xW = None
    for l in range(L):
        W1, b1 = params['enc_W1_%d' % l], params['enc_b1_%d' % l]
        W2, b2 = params['enc_W2_%d' % l], params['enc_b2_%d' % l]
        eps = params['enc_eps_%d' % l]
        hW = h @ W1
        p = _agg_plain(hW, src, dst)
        agg = p[0] + p[1]
        z1 = jnp.maximum((1.0 + eps) * hW + agg + b1, 0.0)
        h = jnp.maximum(z1 @ W2 + b2, 0.0)
        if l == 0:
            xW = hW
    Z = h

    counts = jnp.maximum(scatter(jnp.ones((N, 1), jnp.float32), batch, G), 1.0)
    h_orig = scatter(Z, batch, G) / counts

    nW1cat = jnp.concatenate([params['node_W1'][e] for e in range(NE)], axis=1)
    nb1cat = jnp.concatenate([params['node_b1'][e] for e in range(NE)])
    T = jnp.maximum(Z @ nW1cat + nb1cat, 0.0)
    nw2 = params['node_W2'][:, :, 0].reshape(NE * H)
    nm = jax.nn.sigmoid((T * nw2).reshape(N, NE, H).sum(-1) + params['node_b2'][:, 0])

    eW1a = jnp.concatenate([params['edge_W1'][e, :H, :] for e in range(NE)], axis=1)
    eW1b = jnp.concatenate([params['edge_W1'][e, H:, :] for e in range(NE)], axis=1)
    A = Z @ eW1a
    B = Z @ eW1b
    eb1cat = params['edge_b1'].reshape(NE * H)
    U = jnp.maximum(A[src] + B[dst] + eb1cat, 0.0)
    ew2 = params['edge_W2'][:, :, 0].reshape(NE * H)
    em = jax.nn.sigmoid((U * ew2).reshape(E, NE, H).sum(-1) + params['edge_b2'][:, 0])

    def blockdiag(W):
        return jnp.kron(jnp.eye(NE, dtype=W.dtype), W)

    def to_pair(hcat):
        return hcat.reshape(N, NC, 2 * H).transpose(1, 0, 2)

    def from_pair(p):
        return p.transpose(1, 0, 2).reshape(N, NE * H)

    eps0 = params['enc_eps_0']
    xwnm = jnp.concatenate([xW, nm, jnp.zeros((N, 80 - H - NE), jnp.float32)], axis=1)
    emf = em.reshape(-1)
    agg0 = from_pair(_agg_pair0(xwnm, src, dst, emf))
    self0 = jnp.repeat(nm, H, axis=1) * jnp.tile(xW, (1, NE))
    b1t = jnp.tile(params['enc_b1_0'], NE)
    z1 = jnp.maximum((1.0 + eps0) * self0 + agg0 + b1t, 0.0)
    hm = jnp.maximum(z1 @ blockdiag(params['enc_W2_0']) + jnp.tile(params['enc_b2_0'], NE), 0.0)
    for l in range(1, L):
        W1, b1 = params['enc_W1_%d' % l], params['enc_b1_%d' % l]
        W2, b2 = params['enc_W2_%d' % l], params['enc_b2_%d' % l]
        eps = params['enc_eps_%d' % l]
        hW = hm @ blockdiag(W1)
        agg = from_pair(_agg_pair(to_pair(hW), src, dst, emf))
        z1 = jnp.maximum((1.0 + eps) * hW + agg + jnp.tile(b1, NE), 0.0)
        hm = jnp.maximum(z1 @ blockdiag(W2) + jnp.tile(b2, NE), 0.0)

    h_st = (scatter(hm, batch, G) / counts).reshape(G, NE, H)
    logits = jnp.einsum('geh,ehc->gec', h_st, params['cls_W']) + params['cls_b']
    return (logits, h_st, h_orig, nm[:, :, None], em[:, :, None])
